# Initial kernel scaffold; baseline (speedup 1.0000x reference)
import jax
import jax.numpy as jnp
from jax.experimental import pallas as pl


def kernel(input, table):
    raise NotImplementedError("write your pallas kernel here")


# speedup vs baseline: 3.7638x; 3.7638x over previous
import functools

import jax
import jax.numpy as jnp
from jax import lax
from jax.experimental import pallas as pl
from jax.experimental.pallas import tpu as pltpu
from jax.experimental.pallas import tpu_sc as plsc

NC = 2
NS = 16
NW = NC * NS

CLEN = 128
CPG = 8
GROUP_ROWS = CLEN * CPG


def _make_gather(n_rows, dim, chunks, groups):
    mesh = plsc.VectorSubcoreMesh(core_axis_name="c", subcore_axis_name="s")

    @functools.partial(
        pl.kernel,
        mesh=mesh,
        out_type=jax.ShapeDtypeStruct((NW, groups, GROUP_ROWS, dim),
                                      jnp.float32),
        scratch_types=[
            pltpu.VMEM((chunks, CLEN), jnp.int32),
            pltpu.VMEM((GROUP_ROWS, dim), jnp.float32),
            pltpu.SemaphoreType.DMA,
        ],
        compiler_params=pltpu.CompilerParams(use_tc_tiling_on_sc=False),
    )
    def grab(table_hbm, ids_hbm, out_hbm, idx_v, rows_v, sem):
        wid = lax.axis_index("s") * NC + lax.axis_index("c")
        pltpu.sync_copy(ids_hbm.at[wid], idx_v)

        def body(g):
            handles = []
            for b in range(CPG):
                h = pltpu.async_copy(
                    table_hbm.at[idx_v.at[g * CPG + b]],
                    rows_v.at[pl.ds(b * CLEN, CLEN)],
                    sem,
                )
                handles.append(h)
            for h in handles:
                h.wait()
            pltpu.sync_copy(rows_v, out_hbm.at[wid, g])

        pl.loop(0, groups)(body)

    return grab


def kernel(input, table):
    ids = input
    n = ids.shape[0] * ids.shape[1]
    dim = table.shape[1]
    per_w = n // NW
    chunks = per_w // CLEN
    groups = chunks // CPG
    ids3 = ids.reshape(NW, chunks, CLEN)
    out = _make_gather(table.shape[0], dim, chunks, groups)(table, ids3)
    return out.reshape(ids.shape + (dim,))


# speedup vs baseline: 3.7861x; 1.0059x over previous
import functools

import jax
import jax.numpy as jnp
from jax import lax
from jax.experimental import pallas as pl
from jax.experimental.pallas import tpu as pltpu
from jax.experimental.pallas import tpu_sc as plsc

NC = 2
NS = 16
NW = NC * NS

SLEN = 1024
NBUF = 3


def _make_gather(dim, slots):
    mesh = plsc.VectorSubcoreMesh(core_axis_name="c", subcore_axis_name="s")

    @functools.partial(
        pl.kernel,
        mesh=mesh,
        out_type=jax.ShapeDtypeStruct((NW, slots, SLEN, dim), jnp.float32),
        scratch_types=[
            pltpu.VMEM((slots, SLEN), jnp.int32),
            pltpu.VMEM((NBUF, SLEN, dim), jnp.float32),
            [pltpu.SemaphoreType.DMA] * NBUF,
            [pltpu.SemaphoreType.DMA] * NBUF,
        ],
        compiler_params=pltpu.CompilerParams(use_tc_tiling_on_sc=False),
    )
    def grab(table_hbm, ids_hbm, out_hbm, idx_v, rows_v, gsems, osems):
        wid = lax.axis_index("s") * NC + lax.axis_index("c")
        pltpu.sync_copy(ids_hbm.at[wid], idx_v)

        def fire_gather(g):
            return pltpu.async_copy(
                table_hbm.at[idx_v.at[g]], rows_v.at[g % NBUF], gsems[g % NBUF]
            )

        gh = {g: fire_gather(g) for g in range(min(2, slots))}
        oh = {}
        for g in range(slots):
            gh.pop(g).wait()
            oh[g] = pltpu.async_copy(
                rows_v.at[g % NBUF], out_hbm.at[wid, g], osems[g % NBUF]
            )
            if g + 2 < slots:
                if g - 1 >= 0:
                    oh.pop(g - 1).wait()
                gh[g + 2] = fire_gather(g + 2)
        for h in oh.values():
            h.wait()

    return grab


def kernel(input, table):
    ids = input
    n = ids.shape[0] * ids.shape[1]
    dim = table.shape[1]
    slots = n // (NW * SLEN)
    ids3 = ids.reshape(NW, slots, SLEN)
    out = _make_gather(dim, slots)(table, ids3)
    return out.reshape(ids.shape + (dim,))
